# baseline (speedup 1.0000x reference)
import functools

import jax
import jax.numpy as jnp
from jax import lax
from jax.experimental import pallas as pl
from jax.experimental.pallas import tpu as pltpu
from jax.experimental.pallas import tpu_sc as plsc

_LANES = 16
_NW = 32
_BLK = 128
_LOSS_WEIGHT = 0.1
_UNROLL = 4


def _pick_chunk_blocks(per_w_blocks: int, max_blocks: int = 40) -> int:
    for cb in range(max_blocks, 0, -1):
        if per_w_blocks % cb == 0 and (per_w_blocks // cb) % 2 == 0:
            return cb
    for cb in range(max_blocks, 0, -1):
        if per_w_blocks % cb == 0:
            return cb
    return 1


def _make_sc_partial(n_nodes: int, n_edges: int):
    assert n_edges % _BLK == 0
    nb = n_edges // _BLK
    base_blocks = nb // _NW
    n_extra = nb % _NW
    cb = _pick_chunk_blocks(base_blocks)
    chunk = cb * _BLK
    n_chunks = base_blocks // cb
    groups = chunk // _LANES
    assert groups % _UNROLL == 0
    mesh = plsc.VectorSubcoreMesh(core_axis_name="c", subcore_axis_name="s")

    @functools.partial(
        pl.kernel,
        mesh=mesh,
        compiler_params=pltpu.CompilerParams(needs_layout_passes=False),
        out_type=jax.ShapeDtypeStruct((_NW, _LANES), jnp.float32),
        scratch_types=[
            pltpu.VMEM((n_nodes,), jnp.float32),
            pltpu.VMEM((2, chunk), jnp.int32),
            pltpu.VMEM((2, chunk), jnp.int32),
            pltpu.VMEM((chunk,), jnp.float32),
            pltpu.VMEM((chunk,), jnp.float32),
            pltpu.VMEM((2, _BLK), jnp.int32),
            pltpu.VMEM((_BLK,), jnp.float32),
            pltpu.VMEM((_LANES,), jnp.float32),
            pltpu.SemaphoreType.DMA,
            pltpu.SemaphoreType.DMA,
            pltpu.SemaphoreType.DMA,
            pltpu.SemaphoreType.DMA,
        ],
    )
    def sc_partial(pred_hbm, ei_hbm, w_hbm, out_hbm,
                   pred_v, ei_v0, ei_v1, w_v0, w_v1, ei_tail, w_tail,
                   out_v, psem, sem0, sem1, semt):
        wid = lax.axis_index("s") * 2 + lax.axis_index("c")
        sems = (sem0, sem1)
        ei_v = (ei_v0, ei_v1)
        w_v = (w_v0, w_v1)

        start_blk = wid * base_blocks + jnp.minimum(wid, n_extra)
        has_extra = wid < n_extra
        base = start_blk * _BLK

        pred_cp = pltpu.async_copy(pred_hbm, pred_v, psem)

        def start(b, ci):
            off = pl.multiple_of(base + ci * chunk, _BLK)
            return [
                pltpu.async_copy(ei_hbm.at[:, pl.ds(off, chunk)], ei_v[b], sems[b]),
                pltpu.async_copy(w_hbm.at[pl.ds(off, chunk)], w_v[b], sems[b]),
            ]

        def drain(b):
            pltpu.make_async_copy(ei_hbm.at[:, pl.ds(0, chunk)], ei_v[b], sems[b]).wait()
            pltpu.make_async_copy(w_hbm.at[pl.ds(0, chunk)], w_v[b], sems[b]).wait()

        start(0, 0)
        if n_chunks > 1:
            start(1, 1)
        tail_off = pl.multiple_of(
            jnp.minimum(base + n_chunks * chunk, n_edges - _BLK), _BLK)
        tail_cps = [
            pltpu.async_copy(ei_hbm.at[:, pl.ds(tail_off, _BLK)], ei_tail, semt),
            pltpu.async_copy(w_hbm.at[pl.ds(tail_off, _BLK)], w_tail, semt),
        ]
        pred_cp.wait()

        zero = jnp.zeros((_LANES,), jnp.float32)
        accs = (zero,) * _UNROLL

        def term(eref, wref, g):
            sl = pl.ds(g * _LANES, _LANES)
            return wref[sl] * jnp.abs(
                plsc.load_gather(pred_v, [eref[0, sl]])
                - plsc.load_gather(pred_v, [eref[1, sl]]))

        def compute_chunk(b, accs_in):
            def body(i, a):
                return tuple(
                    a[k] + term(ei_v[b], w_v[b], i + k) for k in range(_UNROLL))

            return plsc.parallel_loop(
                0, groups, step=_UNROLL, carry=accs_in)(body)

        if n_chunks > 1 and n_chunks % 2 == 0:
            def pair(j, accs_in):
                out = accs_in
                for b in range(2):
                    ci = j * 2 + b
                    drain(b)
                    out = compute_chunk(b, out)
                    start(b, jnp.minimum(ci + 2, n_chunks - 1))
                return out

            accs = lax.fori_loop(0, n_chunks // 2, pair, accs)
            drain(0)
            drain(1)
        else:
            for ci in range(n_chunks):
                b = ci % 2
                drain(b)
                accs = compute_chunk(b, accs)
                if ci + 2 < n_chunks:
                    start(b, ci + 2)

        for cp in tail_cps:
            cp.wait()
        tail_sum = zero
        for g in range(_BLK // _LANES):
            tail_sum = tail_sum + term(ei_tail, w_tail, g)
        total = jnp.where(has_extra, tail_sum, zero)
        for a in accs:
            total = total + a

        out_v[...] = total
        pltpu.sync_copy(out_v, out_hbm.at[wid])

    return sc_partial


def kernel(predictions, edge_index, edge_weights):
    n_nodes = predictions.shape[0]
    n_edges = edge_weights.shape[0]
    ei = edge_index.astype(jnp.int32)
    partial = _make_sc_partial(n_nodes, n_edges)(
        predictions, ei, edge_weights)
    return jnp.sum(partial) * (_LOSS_WEIGHT / n_edges)


# speedup vs baseline: 1.0424x; 1.0424x over previous
import functools

import jax
import jax.numpy as jnp
from jax import lax
from jax.experimental import pallas as pl
from jax.experimental.pallas import tpu as pltpu
from jax.experimental.pallas import tpu_sc as plsc

_LANES = 16
_NW = 32
_BLK = 128
_LOSS_WEIGHT = 0.1
_UNROLL = 4


def _pick_chunk_blocks(per_w_blocks: int, max_blocks: int = 80) -> int:
    for cb in range(max_blocks, 0, -1):
        if per_w_blocks % cb == 0:
            return cb
    return 1


def _make_sc_partial(n_nodes: int, n_edges: int):
    assert n_edges % _BLK == 0
    nb = n_edges // _BLK
    base_blocks = nb // _NW
    n_extra = nb % _NW
    cb = _pick_chunk_blocks(base_blocks)
    chunk = cb * _BLK
    n_chunks = base_blocks // cb
    groups = chunk // _LANES
    assert groups % _UNROLL == 0
    mesh = plsc.VectorSubcoreMesh(core_axis_name="c", subcore_axis_name="s")

    @functools.partial(
        pl.kernel,
        mesh=mesh,
        compiler_params=pltpu.CompilerParams(needs_layout_passes=False),
        out_type=jax.ShapeDtypeStruct((_NW, _LANES), jnp.float32),
        scratch_types=[
            pltpu.VMEM((n_nodes,), jnp.float32),
            pltpu.VMEM((2, chunk), jnp.int32),
            pltpu.VMEM((2, chunk), jnp.int32),
            pltpu.VMEM((chunk,), jnp.float32),
            pltpu.VMEM((chunk,), jnp.float32),
            pltpu.VMEM((2, _BLK), jnp.int32),
            pltpu.VMEM((_BLK,), jnp.float32),
            pltpu.VMEM((_LANES,), jnp.float32),
            pltpu.SemaphoreType.DMA,
            pltpu.SemaphoreType.DMA,
            pltpu.SemaphoreType.DMA,
            pltpu.SemaphoreType.DMA,
        ],
    )
    def sc_partial(pred_hbm, ei_hbm, w_hbm, out_hbm,
                   pred_v, ei_v0, ei_v1, w_v0, w_v1, ei_tail, w_tail,
                   out_v, psem, sem0, sem1, semt):
        wid = lax.axis_index("s") * 2 + lax.axis_index("c")
        sems = (sem0, sem1)
        ei_v = (ei_v0, ei_v1)
        w_v = (w_v0, w_v1)

        start_blk = wid * base_blocks + jnp.minimum(wid, n_extra)
        has_extra = wid < n_extra
        base = start_blk * _BLK

        pred_cp = pltpu.async_copy(pred_hbm, pred_v, psem)

        def start(ci):
            b = ci % 2
            off = pl.multiple_of(base + ci * chunk, _BLK)
            return [
                pltpu.async_copy(ei_hbm.at[:, pl.ds(off, chunk)], ei_v[b], sems[b]),
                pltpu.async_copy(w_hbm.at[pl.ds(off, chunk)], w_v[b], sems[b]),
            ]

        inflight = start(0)
        tail_off = pl.multiple_of(
            jnp.minimum(base + n_chunks * chunk, n_edges - _BLK), _BLK)
        tail_cps = [
            pltpu.async_copy(ei_hbm.at[:, pl.ds(tail_off, _BLK)], ei_tail, semt),
            pltpu.async_copy(w_hbm.at[pl.ds(tail_off, _BLK)], w_tail, semt),
        ]
        pred_cp.wait()

        zero = jnp.zeros((_LANES,), jnp.float32)
        accs = (zero,) * _UNROLL

        def term(eref, wref, g):
            sl = pl.ds(g * _LANES, _LANES)
            return wref[sl] * jnp.abs(
                plsc.load_gather(pred_v, [eref[0, sl]])
                - plsc.load_gather(pred_v, [eref[1, sl]]))

        def compute_chunk(b, accs_in):
            def body(i, a):
                return tuple(
                    a[k] + term(ei_v[b], w_v[b], i + k) for k in range(_UNROLL))

            return plsc.parallel_loop(
                0, groups, step=_UNROLL, carry=accs_in)(body)

        for ci in range(n_chunks):
            b = ci % 2
            if ci + 1 < n_chunks:
                nxt = start(ci + 1)
            for cp in inflight:
                cp.wait()
            if ci + 1 < n_chunks:
                inflight = nxt
            accs = compute_chunk(b, accs)

        for cp in tail_cps:
            cp.wait()
        tail_sum = zero
        for g in range(_BLK // _LANES):
            tail_sum = tail_sum + term(ei_tail, w_tail, g)
        total = jnp.where(has_extra, tail_sum, zero)
        for a in accs:
            total = total + a

        out_v[...] = total
        pltpu.sync_copy(out_v, out_hbm.at[wid])

    return sc_partial


def kernel(predictions, edge_index, edge_weights):
    n_nodes = predictions.shape[0]
    n_edges = edge_weights.shape[0]
    ei = edge_index.astype(jnp.int32)
    partial = _make_sc_partial(n_nodes, n_edges)(
        predictions, ei, edge_weights)
    return jnp.sum(partial) * (_LOSS_WEIGHT / n_edges)
